# baseline (speedup 1.0000x reference)
import jax
import jax.numpy as jnp
from jax import lax
from jax.experimental import pallas as pl
from jax.experimental.pallas import tpu as pltpu
from jax.experimental.pallas import tpu_sc as plsc

NN = 10000
NE = 320000
DD = 128
DH = DD // 2
NG = 64

NPAD = 10240
EROWS = 2560
CHUNK = 128
ROWS_PER_TILE = EROWS // 16
STRIPE = NPAD // 16
SUB = STRIPE // CHUNK


IBLK = 16


def _zero_rows(rows):
    def fill_rows(t, carry):
        rows[t // (DH // 16), pl.ds((t % (DH // 16)) * 16, 16)] = (
            jnp.zeros((16,), jnp.float32))
        return carry

    lax.fori_loop(0, CHUNK * (DH // 16), fill_rows, 0)


def _edge_loop(h, srcv, dstv, rows0, rows1, sem0, sem1, ssem0, ssem1,
               acc, cacc, onesv, srcr, dstr, sid, with_counts):
    rows = (rows0, rows1)
    sems = (sem0, sem1)
    ssems = (ssem0, ssem1)

    def outer(jo, carry):
        eb = pl.multiple_of(sid * ROWS_PER_TILE + jo * IBLK, 8)
        pltpu.sync_copy(srcr.at[pl.ds(eb, IBLK)], srcv)
        pltpu.sync_copy(dstr.at[pl.ds(eb, IBLK)], dstv)
        desc = [None, None]
        scat = [None, None]
        desc[0] = pltpu.async_copy(h.at[srcv.at[0]], rows[0], sems[0])
        for t in range(IBLK):
            b = t & 1
            if t + 1 < IBLK:
                if scat[1 - b] is not None:
                    scat[1 - b].wait()
                desc[1 - b] = pltpu.async_copy(
                    h.at[srcv.at[t + 1]], rows[1 - b], sems[1 - b])
            desc[b].wait()
            scat[b] = None
            if with_counts:
                pltpu.sync_copy(onesv, cacc.at[dstv.at[t]], add=True)
        return carry

    lax.fori_loop(0, ROWS_PER_TILE // IBLK, outer, 0)


def _agg_body_c(h0, h1, srcr, dstr, agg0o, agg1o, cnto,
                acc, cacc, srcv, dstv, rows0, rows1, onesv, z16,
                sem0, sem1, ssem0, ssem1):
    cid = lax.axis_index("c")
    sid = lax.axis_index("s")
    r0 = sid * STRIPE

    _zero_rows(rows0)

    def fill_16(i, carry):
        onesv[i, :] = jnp.ones((16,), jnp.float32)
        return carry

    lax.fori_loop(0, CHUNK, fill_16, 0)

    def fill_z16(i, carry):
        z16[i, :] = jnp.zeros((16,), jnp.float32)
        return carry

    lax.fori_loop(0, 64, fill_z16, 0)

    for k in range(SUB):
        pltpu.sync_copy(rows0, acc.at[pl.ds(r0 + k * CHUNK, CHUNK)])
    for k in range(STRIPE // 64):
        pltpu.sync_copy(z16, cacc.at[pl.ds(r0 + k * 64, 64)])

    plsc.subcore_barrier()

    @pl.when(cid == 0)
    def _():
        _edge_loop(h0, srcv, dstv, rows0, rows1, sem0, sem1, ssem0, ssem1,
                   acc, cacc, onesv, srcr, dstr, sid, True)

    @pl.when(cid == 1)
    def _():
        _edge_loop(h1, srcv, dstv, rows0, rows1, sem0, sem1, ssem0, ssem1,
                   acc, cacc, onesv, srcr, dstr, sid, False)

    plsc.subcore_barrier()

    for k in range(SUB):
        rk = r0 + k * CHUNK
        pltpu.sync_copy(acc.at[pl.ds(rk, CHUNK)], rows0)

        @pl.when(cid == 0)
        def _():
            pltpu.sync_copy(rows0, agg0o.at[pl.ds(rk, CHUNK)])
            pltpu.sync_copy(cacc.at[pl.ds(rk, CHUNK)], onesv)
            pltpu.sync_copy(onesv, cnto.at[pl.ds(rk, CHUNK)])

        @pl.when(cid == 1)
        def _():
            pltpu.sync_copy(rows0, agg1o.at[pl.ds(rk, CHUNK)])


def _agg_body_nc(hf, h1, srcr, dstr, agg0o, agg1o,
                 acc, srcv, dstv, rows0, rows1, sem0, sem1, ssem0, ssem1):
    cid = lax.axis_index("c")
    sid = lax.axis_index("s")
    wid = cid * 16 + sid
    r0 = sid * STRIPE

    plsc.subcore_barrier()

    def outer(jo, carry):
        eb = pl.multiple_of(wid * (ROWS_PER_TILE // 2) + jo * IBLK, 8)
        pltpu.sync_copy(srcr.at[pl.ds(eb, IBLK)], srcv)
        desc = [None, None]
        desc[0] = pltpu.async_copy(hf.at[srcv.at[0]], rows0, sem0)
        rows = (rows0, rows1)
        sems = (sem0, sem1)
        for t in range(IBLK):
            b = t & 1
            if t + 1 < IBLK:
                desc[1 - b] = pltpu.async_copy(
                    hf.at[srcv.at[t + 1]], rows[1 - b], sems[1 - b])
            desc[b].wait()
        return carry

    lax.fori_loop(0, (ROWS_PER_TILE // 2) // IBLK, outer, 0)

    plsc.subcore_barrier()


_SC_MESH = plsc.VectorSubcoreMesh(core_axis_name="c", subcore_axis_name="s")
_SC_PARAMS = pltpu.CompilerParams(use_tc_tiling_on_sc=False)

_agg_c = pl.kernel(
    _agg_body_c,
    out_type=(
        jax.ShapeDtypeStruct((NPAD, DH), jnp.float32),
        jax.ShapeDtypeStruct((NPAD, DH), jnp.float32),
        jax.ShapeDtypeStruct((NPAD, 16), jnp.float32),
    ),
    mesh=_SC_MESH,
    compiler_params=_SC_PARAMS,
    scratch_types=(
        pltpu.VMEM_SHARED((NPAD, DH), jnp.float32),
        pltpu.VMEM_SHARED((NPAD, 16), jnp.float32),
        pltpu.VMEM((IBLK, CHUNK), jnp.int32),
        pltpu.VMEM((IBLK, CHUNK), jnp.int32),
        pltpu.VMEM((CHUNK, DH), jnp.float32),
        pltpu.VMEM((CHUNK, DH), jnp.float32),
        pltpu.VMEM((CHUNK, 16), jnp.float32),
        pltpu.VMEM((64, 16), jnp.float32),
        pltpu.SemaphoreType.DMA,
        pltpu.SemaphoreType.DMA,
        pltpu.SemaphoreType.DMA,
        pltpu.SemaphoreType.DMA,
    ),
)

_agg_nc = pl.kernel(
    _agg_body_nc,
    out_type=(
        jax.ShapeDtypeStruct((NPAD, DH), jnp.float32),
        jax.ShapeDtypeStruct((NPAD, DH), jnp.float32),
    ),
    mesh=_SC_MESH,
    compiler_params=_SC_PARAMS,
    scratch_types=(
        pltpu.VMEM_SHARED((NPAD, DH), jnp.float32),
        pltpu.VMEM((IBLK, CHUNK), jnp.int32),
        pltpu.VMEM((IBLK, CHUNK), jnp.int32),
        pltpu.VMEM((CHUNK, DD), jnp.float32),
        pltpu.VMEM((CHUNK, DD), jnp.float32),
        pltpu.SemaphoreType.DMA,
        pltpu.SemaphoreType.DMA,
        pltpu.SemaphoreType.DMA,
        pltpu.SemaphoreType.DMA,
    ),
)


def _dense_body(a0_ref, a1_ref, cnt_ref, h0_ref, h1_ref, wl_ref, wr_ref,
                b_ref, o0_ref, o1_ref):
    rc = 1.0 / jnp.maximum(cnt_ref[:, 0:1], 1.0)
    mean = jnp.concatenate([a0_ref[...], a1_ref[...]], axis=1) * rc
    h = jnp.concatenate([h0_ref[...], h1_ref[...]], axis=1)
    o = jnp.dot(mean, wl_ref[...], preferred_element_type=jnp.float32)
    o = o + jnp.dot(h, wr_ref[...], preferred_element_type=jnp.float32)
    o = jnp.maximum(o + b_ref[...], 0.0)
    o0_ref[...] = o[:, :DH]
    o1_ref[...] = o[:, DH:]


_dense = pl.pallas_call(
    _dense_body,
    grid=(10,),
    in_specs=[
        pl.BlockSpec((1024, DH), lambda i: (i, 0)),
        pl.BlockSpec((1024, DH), lambda i: (i, 0)),
        pl.BlockSpec((1024, 16), lambda i: (i, 0)),
        pl.BlockSpec((1024, DH), lambda i: (i, 0)),
        pl.BlockSpec((1024, DH), lambda i: (i, 0)),
        pl.BlockSpec((DD, DD), lambda i: (0, 0)),
        pl.BlockSpec((DD, DD), lambda i: (0, 0)),
        pl.BlockSpec((1, DD), lambda i: (0, 0)),
    ],
    out_specs=[
        pl.BlockSpec((1024, DH), lambda i: (i, 0)),
        pl.BlockSpec((1024, DH), lambda i: (i, 0)),
    ],
    out_shape=[
        jax.ShapeDtypeStruct((NN, DH), jnp.float32),
        jax.ShapeDtypeStruct((NN, DH), jnp.float32),
    ],
)


def _pool_body(h0_ref, h1_ref, batch_ref, wc1_ref, bc1_ref, wc2_ref,
               bc2_ref, o_ref):
    h = jnp.concatenate([h0_ref[...], h1_ref[...]], axis=1)
    gids = lax.broadcasted_iota(jnp.int32, (NG, NN), 0)
    sel = jnp.where(batch_ref[...] == gids, 1.0, 0.0)
    cnts = jnp.sum(sel, axis=1, keepdims=True)
    ps = jnp.dot(sel, h, preferred_element_type=jnp.float32,
                 precision=lax.Precision.HIGHEST)
    pooled = ps / jnp.maximum(cnts, 1.0)
    z = jnp.dot(pooled, wc1_ref[...], preferred_element_type=jnp.float32)
    z = jnp.maximum(z + bc1_ref[...], 0.0)
    o_ref[...] = jnp.dot(z, wc2_ref[...], preferred_element_type=jnp.float32) + bc2_ref[...]


_pool = pl.pallas_call(
    _pool_body,
    out_shape=jax.ShapeDtypeStruct((NG, DD), jnp.float32),
)


def kernel(x, edge_index, batch, W1l, W1r, b1, W2l, W2r, b2, W3l, W3r, b3,
           Wc1, bc1, Wc2, bc2):
    src = edge_index[0].astype(jnp.int32)
    dst = edge_index[1].astype(jnp.int32)
    npad = EROWS * CHUNK - NE
    src = jnp.concatenate([src, jnp.zeros((npad,), jnp.int32)]).reshape(EROWS, CHUNK)
    dst = jnp.concatenate([dst, jnp.full((npad,), NN, jnp.int32)]).reshape(EROWS, CHUNK)

    h0 = x[:, :DH]
    h1 = x[:, DH:]
    a0, a1, cnt = _agg_c(h0, h1, src, dst)
    h0, h1 = _dense(a0, a1, cnt, h0, h1, W1l, W1r, b1.reshape(1, DD))
    for Wl, Wr, b in ((W2l, W2r, b2), (W3l, W3r, b3)):
        hf = jnp.concatenate([h0, h1], axis=1)
        a0, a1 = _agg_nc(hf, h1, src, dst)
        h0, h1 = _dense(a0, a1, cnt, h0, h1, Wl, Wr, b.reshape(1, DD))

    batch32 = batch.astype(jnp.int32).reshape(1, NN)
    wc2p = jnp.pad(Wc2, ((0, 0), (0, DD - 2)))
    bc2p = jnp.pad(bc2, (0, DD - 2)).reshape(1, DD)
    out = _pool(h0, h1, batch32, Wc1, bc1.reshape(1, DD // 2), wc2p, bc2p)
    return out[:, :2]


# speedup vs baseline: 1.2054x; 1.2054x over previous
import jax
import jax.numpy as jnp
from jax import lax
from jax.experimental import pallas as pl
from jax.experimental.pallas import tpu as pltpu
from jax.experimental.pallas import tpu_sc as plsc

NN = 10000
NE = 320000
DD = 128
DH = DD // 2
NG = 64

NPAD = 10240
EROWS = 2560
CHUNK = 128
ROWS_PER_TILE = EROWS // 16
STRIPE = NPAD // 16
SUB = STRIPE // CHUNK


IBLK = 16


def _zero_rows(rows):
    def fill_rows(t, carry):
        rows[t // (DH // 16), pl.ds((t % (DH // 16)) * 16, 16)] = (
            jnp.zeros((16,), jnp.float32))
        return carry

    lax.fori_loop(0, CHUNK * (DH // 16), fill_rows, 0)


def _edge_loop(h, srcv, dstv, rows0, rows1, sem0, sem1, ssem0, ssem1,
               acc, cacc, onesv, srcr, dstr, sid, with_counts):
    rows = (rows0, rows1)
    sems = (sem0, sem1)
    ssems = (ssem0, ssem1)

    def outer(jo, carry):
        eb = pl.multiple_of(sid * ROWS_PER_TILE + jo * IBLK, 8)
        pltpu.sync_copy(srcr.at[pl.ds(eb, IBLK)], srcv)
        pltpu.sync_copy(dstr.at[pl.ds(eb, IBLK)], dstv)
        desc = [None, None]
        scat = [None, None]
        desc[0] = pltpu.async_copy(h.at[srcv.at[0]], rows[0], sems[0])
        for t in range(IBLK):
            b = t & 1
            if t + 1 < IBLK:
                if scat[1 - b] is not None:
                    scat[1 - b].wait()
                desc[1 - b] = pltpu.async_copy(
                    h.at[srcv.at[t + 1]], rows[1 - b], sems[1 - b])
            desc[b].wait()
            scat[b] = pltpu.async_copy(
                rows[b], acc.at[dstv.at[t]], ssems[b], add=True)
            if with_counts:
                pltpu.sync_copy(onesv, cacc.at[dstv.at[t]], add=True)
        for b in range(2):
            if scat[b] is not None:
                scat[b].wait()
        return carry

    lax.fori_loop(0, ROWS_PER_TILE // IBLK, outer, 0)


def _agg_body_c(h0, h1, srcr, dstr, agg0o, agg1o, cnto,
                acc, cacc, srcv, dstv, rows0, rows1, onesv, z16,
                sem0, sem1, ssem0, ssem1):
    cid = lax.axis_index("c")
    sid = lax.axis_index("s")
    r0 = sid * STRIPE

    _zero_rows(rows0)

    def fill_16(i, carry):
        onesv[i, :] = jnp.ones((16,), jnp.float32)
        return carry

    lax.fori_loop(0, CHUNK, fill_16, 0)

    def fill_z16(i, carry):
        z16[i, :] = jnp.zeros((16,), jnp.float32)
        return carry

    lax.fori_loop(0, 64, fill_z16, 0)

    for k in range(SUB):
        pltpu.sync_copy(rows0, acc.at[pl.ds(r0 + k * CHUNK, CHUNK)])
    for k in range(STRIPE // 64):
        pltpu.sync_copy(z16, cacc.at[pl.ds(r0 + k * 64, 64)])

    plsc.subcore_barrier()

    @pl.when(cid == 0)
    def _():
        _edge_loop(h0, srcv, dstv, rows0, rows1, sem0, sem1, ssem0, ssem1,
                   acc, cacc, onesv, srcr, dstr, sid, True)

    @pl.when(cid == 1)
    def _():
        _edge_loop(h1, srcv, dstv, rows0, rows1, sem0, sem1, ssem0, ssem1,
                   acc, cacc, onesv, srcr, dstr, sid, False)

    plsc.subcore_barrier()

    for k in range(SUB):
        rk = r0 + k * CHUNK
        pltpu.sync_copy(acc.at[pl.ds(rk, CHUNK)], rows0)

        @pl.when(cid == 0)
        def _():
            pltpu.sync_copy(rows0, agg0o.at[pl.ds(rk, CHUNK)])
            pltpu.sync_copy(cacc.at[pl.ds(rk, CHUNK)], onesv)
            pltpu.sync_copy(onesv, cnto.at[pl.ds(rk, CHUNK)])

        @pl.when(cid == 1)
        def _():
            pltpu.sync_copy(rows0, agg1o.at[pl.ds(rk, CHUNK)])


def _agg_body_nc(h0, h1, srcr, dstr, agg0o, agg1o,
                 acc, srcv, dstv, rows0, rows1, sem0, sem1, ssem0, ssem1):
    cid = lax.axis_index("c")
    sid = lax.axis_index("s")
    r0 = sid * STRIPE

    _zero_rows(rows0)

    for k in range(SUB):
        pltpu.sync_copy(rows0, acc.at[pl.ds(r0 + k * CHUNK, CHUNK)])

    plsc.subcore_barrier()

    @pl.when(cid == 0)
    def _():
        _edge_loop(h0, srcv, dstv, rows0, rows1, sem0, sem1, ssem0, ssem1,
                   acc, None, None, srcr, dstr, sid, False)

    @pl.when(cid == 1)
    def _():
        _edge_loop(h1, srcv, dstv, rows0, rows1, sem0, sem1, ssem0, ssem1,
                   acc, None, None, srcr, dstr, sid, False)

    plsc.subcore_barrier()

    for k in range(SUB):
        rk = r0 + k * CHUNK
        pltpu.sync_copy(acc.at[pl.ds(rk, CHUNK)], rows0)

        @pl.when(cid == 0)
        def _():
            pltpu.sync_copy(rows0, agg0o.at[pl.ds(rk, CHUNK)])

        @pl.when(cid == 1)
        def _():
            pltpu.sync_copy(rows0, agg1o.at[pl.ds(rk, CHUNK)])


_SC_MESH = plsc.VectorSubcoreMesh(core_axis_name="c", subcore_axis_name="s")
_SC_PARAMS = pltpu.CompilerParams(use_tc_tiling_on_sc=False)

_agg_c = pl.kernel(
    _agg_body_c,
    out_type=(
        jax.ShapeDtypeStruct((NPAD, DH), jnp.float32),
        jax.ShapeDtypeStruct((NPAD, DH), jnp.float32),
        jax.ShapeDtypeStruct((NPAD, 16), jnp.float32),
    ),
    mesh=_SC_MESH,
    compiler_params=_SC_PARAMS,
    scratch_types=(
        pltpu.VMEM_SHARED((NPAD, DH), jnp.float32),
        pltpu.VMEM_SHARED((NPAD, 16), jnp.float32),
        pltpu.VMEM((IBLK, CHUNK), jnp.int32),
        pltpu.VMEM((IBLK, CHUNK), jnp.int32),
        pltpu.VMEM((CHUNK, DH), jnp.float32),
        pltpu.VMEM((CHUNK, DH), jnp.float32),
        pltpu.VMEM((CHUNK, 16), jnp.float32),
        pltpu.VMEM((64, 16), jnp.float32),
        pltpu.SemaphoreType.DMA,
        pltpu.SemaphoreType.DMA,
        pltpu.SemaphoreType.DMA,
        pltpu.SemaphoreType.DMA,
    ),
)

_agg_nc = pl.kernel(
    _agg_body_nc,
    out_type=(
        jax.ShapeDtypeStruct((NPAD, DH), jnp.float32),
        jax.ShapeDtypeStruct((NPAD, DH), jnp.float32),
    ),
    mesh=_SC_MESH,
    compiler_params=_SC_PARAMS,
    scratch_types=(
        pltpu.VMEM_SHARED((NPAD, DH), jnp.float32),
        pltpu.VMEM((IBLK, CHUNK), jnp.int32),
        pltpu.VMEM((IBLK, CHUNK), jnp.int32),
        pltpu.VMEM((CHUNK, DH), jnp.float32),
        pltpu.VMEM((CHUNK, DH), jnp.float32),
        pltpu.SemaphoreType.DMA,
        pltpu.SemaphoreType.DMA,
        pltpu.SemaphoreType.DMA,
        pltpu.SemaphoreType.DMA,
    ),
)


def _dense_body(a0_ref, a1_ref, cnt_ref, h0_ref, h1_ref, wl_ref, wr_ref,
                b_ref, o0_ref, o1_ref):
    rc = 1.0 / jnp.maximum(cnt_ref[:, 0:1], 1.0)
    mean = jnp.concatenate([a0_ref[...], a1_ref[...]], axis=1) * rc
    h = jnp.concatenate([h0_ref[...], h1_ref[...]], axis=1)
    o = jnp.dot(mean, wl_ref[...], preferred_element_type=jnp.float32)
    o = o + jnp.dot(h, wr_ref[...], preferred_element_type=jnp.float32)
    o = jnp.maximum(o + b_ref[...], 0.0)
    o0_ref[...] = o[:, :DH]
    o1_ref[...] = o[:, DH:]


_dense = pl.pallas_call(
    _dense_body,
    grid=(10,),
    in_specs=[
        pl.BlockSpec((1024, DH), lambda i: (i, 0)),
        pl.BlockSpec((1024, DH), lambda i: (i, 0)),
        pl.BlockSpec((1024, 16), lambda i: (i, 0)),
        pl.BlockSpec((1024, DH), lambda i: (i, 0)),
        pl.BlockSpec((1024, DH), lambda i: (i, 0)),
        pl.BlockSpec((DD, DD), lambda i: (0, 0)),
        pl.BlockSpec((DD, DD), lambda i: (0, 0)),
        pl.BlockSpec((1, DD), lambda i: (0, 0)),
    ],
    out_specs=[
        pl.BlockSpec((1024, DH), lambda i: (i, 0)),
        pl.BlockSpec((1024, DH), lambda i: (i, 0)),
    ],
    out_shape=[
        jax.ShapeDtypeStruct((NN, DH), jnp.float32),
        jax.ShapeDtypeStruct((NN, DH), jnp.float32),
    ],
)


def _pool_body(h0_ref, h1_ref, batch_ref, wc1_ref, bc1_ref, wc2_ref,
               bc2_ref, o_ref):
    h = jnp.concatenate([h0_ref[...], h1_ref[...]], axis=1)
    gids = lax.broadcasted_iota(jnp.int32, (NG, NN), 0)
    sel = jnp.where(batch_ref[...] == gids, 1.0, 0.0)
    cnts = jnp.sum(sel, axis=1, keepdims=True)
    ps = jnp.dot(sel, h, preferred_element_type=jnp.float32,
                 precision=lax.Precision.HIGHEST)
    pooled = ps / jnp.maximum(cnts, 1.0)
    z = jnp.dot(pooled, wc1_ref[...], preferred_element_type=jnp.float32)
    z = jnp.maximum(z + bc1_ref[...], 0.0)
    o_ref[...] = jnp.dot(z, wc2_ref[...], preferred_element_type=jnp.float32) + bc2_ref[...]


_pool = pl.pallas_call(
    _pool_body,
    out_shape=jax.ShapeDtypeStruct((NG, DD), jnp.float32),
)


def kernel(x, edge_index, batch, W1l, W1r, b1, W2l, W2r, b2, W3l, W3r, b3,
           Wc1, bc1, Wc2, bc2):
    src = edge_index[0].astype(jnp.int32)
    dst = edge_index[1].astype(jnp.int32)
    npad = EROWS * CHUNK - NE
    src = jnp.concatenate([src, jnp.zeros((npad,), jnp.int32)]).reshape(EROWS, CHUNK)
    dst = jnp.concatenate([dst, jnp.full((npad,), NN, jnp.int32)]).reshape(EROWS, CHUNK)

    h0 = x[:, :DH]
    h1 = x[:, DH:]
    a0, a1, cnt = _agg_c(h0, h1, src, dst)
    h0, h1 = _dense(a0, a1, cnt, h0, h1, W1l, W1r, b1.reshape(1, DD))
    for Wl, Wr, b in ((W2l, W2r, b2), (W3l, W3r, b3)):
        a0, a1 = _agg_nc(h0, h1, src, dst)
        h0, h1 = _dense(a0, a1, cnt, h0, h1, Wl, Wr, b.reshape(1, DD))

    batch32 = batch.astype(jnp.int32).reshape(1, NN)
    wc2p = jnp.pad(Wc2, ((0, 0), (0, DD - 2)))
    bc2p = jnp.pad(bc2, (0, DD - 2)).reshape(1, DD)
    out = _pool(h0, h1, batch32, Wc1, bc1.reshape(1, DD // 2), wc2p, bc2p)
    return out[:, :2]


# speedup vs baseline: 1.6923x; 1.4039x over previous
import jax
import jax.numpy as jnp
from jax import lax
from jax.experimental import pallas as pl
from jax.experimental.pallas import tpu as pltpu
from jax.experimental.pallas import tpu_sc as plsc

NN = 10000
NE = 320000
DD = 128
DH = DD // 2
NG = 64

NPAD = 10240
EROWS = 2560
CHUNK = 128
ROWS_PER_TILE = EROWS // 16
STRIPE = NPAD // 16
SUB = STRIPE // CHUNK


IBLK = 16


def _zero_rows(rows):
    def fill_rows(t, carry):
        rows[t // (DH // 16), pl.ds((t % (DH // 16)) * 16, 16)] = (
            jnp.zeros((16,), jnp.float32))
        return carry

    lax.fori_loop(0, CHUNK * (DH // 16), fill_rows, 0)


def _edge_loop(h, srcv, dstv, rows0, rows1, sem0, sem1, ssem0, ssem1,
               acc, cacc, onesv, srcr, dstr, sid, with_counts):
    rows = (rows0, rows1)
    sems = (sem0, sem1)
    ssems = (ssem0, ssem1)

    def outer(jo, carry):
        eb = pl.multiple_of(sid * ROWS_PER_TILE + jo * IBLK, 8)
        pltpu.sync_copy(srcr.at[pl.ds(eb, IBLK)], srcv)
        pltpu.sync_copy(dstr.at[pl.ds(eb, IBLK)], dstv)
        desc = [None, None]
        scat = [None, None]
        desc[0] = pltpu.async_copy(h.at[srcv.at[0]], rows[0], sems[0])
        for t in range(IBLK):
            b = t & 1
            if t + 1 < IBLK:
                if scat[1 - b] is not None:
                    scat[1 - b].wait()
                desc[1 - b] = pltpu.async_copy(
                    h.at[srcv.at[t + 1]], rows[1 - b], sems[1 - b])
            desc[b].wait()
            scat[b] = pltpu.async_copy(
                rows[b], acc.at[dstv.at[t]], ssems[b], add=True)
            if with_counts:
                pltpu.sync_copy(onesv, cacc.at[dstv.at[t]], add=True)
        for b in range(2):
            if scat[b] is not None:
                scat[b].wait()
        return carry

    lax.fori_loop(0, ROWS_PER_TILE // IBLK, outer, 0)


def _agg_body_c(h0, h1, srcr, dstr, agg0o, agg1o, cnto,
                acc, cacc, srcv, dstv, rows0, rows1, onesv, z16,
                sem0, sem1, ssem0, ssem1):
    cid = lax.axis_index("c")
    sid = lax.axis_index("s")
    r0 = sid * STRIPE

    _zero_rows(rows0)

    def fill_16(i, carry):
        onesv[i, :] = jnp.ones((16,), jnp.float32)
        return carry

    lax.fori_loop(0, CHUNK, fill_16, 0)

    def fill_z16(i, carry):
        z16[i, :] = jnp.zeros((16,), jnp.float32)
        return carry

    lax.fori_loop(0, 64, fill_z16, 0)

    for k in range(SUB):
        pltpu.sync_copy(rows0, acc.at[pl.ds(r0 + k * CHUNK, CHUNK)])
    for k in range(STRIPE // 64):
        pltpu.sync_copy(z16, cacc.at[pl.ds(r0 + k * 64, 64)])

    plsc.subcore_barrier()

    @pl.when(cid == 0)
    def _():
        _edge_loop(h0, srcv, dstv, rows0, rows1, sem0, sem1, ssem0, ssem1,
                   acc, cacc, onesv, srcr, dstr, sid, True)

    @pl.when(cid == 1)
    def _():
        _edge_loop(h1, srcv, dstv, rows0, rows1, sem0, sem1, ssem0, ssem1,
                   acc, cacc, onesv, srcr, dstr, sid, False)

    plsc.subcore_barrier()

    for k in range(SUB):
        rk = r0 + k * CHUNK
        pltpu.sync_copy(acc.at[pl.ds(rk, CHUNK)], rows0)

        @pl.when(cid == 0)
        def _():
            pltpu.sync_copy(rows0, agg0o.at[pl.ds(rk, CHUNK)])
            pltpu.sync_copy(cacc.at[pl.ds(rk, CHUNK)], onesv)
            pltpu.sync_copy(onesv, cnto.at[pl.ds(rk, CHUNK)])

        @pl.when(cid == 1)
        def _():
            pltpu.sync_copy(rows0, agg1o.at[pl.ds(rk, CHUNK)])


def _agg_body_nc(h0, h1, srcr, dstr, agg0o, agg1o,
                 hsp, acc, srcv, dstv, rows0, rows1, sem0, sem1, ssem0, ssem1):
    cid = lax.axis_index("c")
    sid = lax.axis_index("s")
    r0 = sid * STRIPE

    _zero_rows(rows0)

    for k in range(SUB):
        pltpu.sync_copy(rows0, acc.at[pl.ds(r0 + k * CHUNK, CHUNK)])

    @pl.when(cid == 0)
    def _():
        pltpu.sync_copy(h0.at[pl.ds(r0, STRIPE)], hsp.at[pl.ds(r0, STRIPE)])

    @pl.when(cid == 1)
    def _():
        pltpu.sync_copy(h1.at[pl.ds(r0, STRIPE)], hsp.at[pl.ds(r0, STRIPE)])

    plsc.subcore_barrier()

    _edge_loop(hsp, srcv, dstv, rows0, rows1, sem0, sem1, ssem0, ssem1,
               acc, None, None, srcr, dstr, sid, False)

    plsc.subcore_barrier()

    for k in range(SUB):
        rk = r0 + k * CHUNK
        pltpu.sync_copy(acc.at[pl.ds(rk, CHUNK)], rows0)

        @pl.when(cid == 0)
        def _():
            pltpu.sync_copy(rows0, agg0o.at[pl.ds(rk, CHUNK)])

        @pl.when(cid == 1)
        def _():
            pltpu.sync_copy(rows0, agg1o.at[pl.ds(rk, CHUNK)])


_SC_MESH = plsc.VectorSubcoreMesh(core_axis_name="c", subcore_axis_name="s")
_SC_PARAMS = pltpu.CompilerParams(use_tc_tiling_on_sc=False)

_agg_c = pl.kernel(
    _agg_body_c,
    out_type=(
        jax.ShapeDtypeStruct((NPAD, DH), jnp.float32),
        jax.ShapeDtypeStruct((NPAD, DH), jnp.float32),
        jax.ShapeDtypeStruct((NPAD, 16), jnp.float32),
    ),
    mesh=_SC_MESH,
    compiler_params=_SC_PARAMS,
    scratch_types=(
        pltpu.VMEM_SHARED((NPAD, DH), jnp.float32),
        pltpu.VMEM_SHARED((NPAD, 16), jnp.float32),
        pltpu.VMEM((IBLK, CHUNK), jnp.int32),
        pltpu.VMEM((IBLK, CHUNK), jnp.int32),
        pltpu.VMEM((CHUNK, DH), jnp.float32),
        pltpu.VMEM((CHUNK, DH), jnp.float32),
        pltpu.VMEM((CHUNK, 16), jnp.float32),
        pltpu.VMEM((64, 16), jnp.float32),
        pltpu.SemaphoreType.DMA,
        pltpu.SemaphoreType.DMA,
        pltpu.SemaphoreType.DMA,
        pltpu.SemaphoreType.DMA,
    ),
)

_agg_nc = pl.kernel(
    _agg_body_nc,
    out_type=(
        jax.ShapeDtypeStruct((NPAD, DH), jnp.float32),
        jax.ShapeDtypeStruct((NPAD, DH), jnp.float32),
    ),
    mesh=_SC_MESH,
    compiler_params=_SC_PARAMS,
    scratch_types=(
        pltpu.VMEM_SHARED((NPAD, DH), jnp.float32),
        pltpu.VMEM_SHARED((NPAD, DH), jnp.float32),
        pltpu.VMEM((IBLK, CHUNK), jnp.int32),
        pltpu.VMEM((IBLK, CHUNK), jnp.int32),
        pltpu.VMEM((CHUNK, DH), jnp.float32),
        pltpu.VMEM((CHUNK, DH), jnp.float32),
        pltpu.SemaphoreType.DMA,
        pltpu.SemaphoreType.DMA,
        pltpu.SemaphoreType.DMA,
        pltpu.SemaphoreType.DMA,
    ),
)


def _dense_body(a0_ref, a1_ref, cnt_ref, h0_ref, h1_ref, wl_ref, wr_ref,
                b_ref, o0_ref, o1_ref):
    rc = 1.0 / jnp.maximum(cnt_ref[:, 0:1], 1.0)
    mean = jnp.concatenate([a0_ref[...], a1_ref[...]], axis=1) * rc
    h = jnp.concatenate([h0_ref[...], h1_ref[...]], axis=1)
    o = jnp.dot(mean, wl_ref[...], preferred_element_type=jnp.float32)
    o = o + jnp.dot(h, wr_ref[...], preferred_element_type=jnp.float32)
    o = jnp.maximum(o + b_ref[...], 0.0)
    o0_ref[...] = o[:, :DH]
    o1_ref[...] = o[:, DH:]


_dense = pl.pallas_call(
    _dense_body,
    grid=(10,),
    in_specs=[
        pl.BlockSpec((1024, DH), lambda i: (i, 0)),
        pl.BlockSpec((1024, DH), lambda i: (i, 0)),
        pl.BlockSpec((1024, 16), lambda i: (i, 0)),
        pl.BlockSpec((1024, DH), lambda i: (i, 0)),
        pl.BlockSpec((1024, DH), lambda i: (i, 0)),
        pl.BlockSpec((DD, DD), lambda i: (0, 0)),
        pl.BlockSpec((DD, DD), lambda i: (0, 0)),
        pl.BlockSpec((1, DD), lambda i: (0, 0)),
    ],
    out_specs=[
        pl.BlockSpec((1024, DH), lambda i: (i, 0)),
        pl.BlockSpec((1024, DH), lambda i: (i, 0)),
    ],
    out_shape=[
        jax.ShapeDtypeStruct((NPAD, DH), jnp.float32),
        jax.ShapeDtypeStruct((NPAD, DH), jnp.float32),
    ],
)


def _pool_body(h0_ref, h1_ref, batch_ref, wc1_ref, bc1_ref, wc2_ref,
               bc2_ref, o_ref):
    h = jnp.concatenate([h0_ref[...], h1_ref[...]], axis=1)
    gids = lax.broadcasted_iota(jnp.int32, (NG, NPAD), 0)
    sel = jnp.where(batch_ref[...] == gids, 1.0, 0.0)
    cnts = jnp.sum(sel, axis=1, keepdims=True)
    ps = jnp.dot(sel, h, preferred_element_type=jnp.float32,
                 precision=lax.Precision.HIGHEST)
    pooled = ps / jnp.maximum(cnts, 1.0)
    z = jnp.dot(pooled, wc1_ref[...], preferred_element_type=jnp.float32)
    z = jnp.maximum(z + bc1_ref[...], 0.0)
    o_ref[...] = jnp.dot(z, wc2_ref[...], preferred_element_type=jnp.float32) + bc2_ref[...]


_pool = pl.pallas_call(
    _pool_body,
    out_shape=jax.ShapeDtypeStruct((NG, DD), jnp.float32),
)


def kernel(x, edge_index, batch, W1l, W1r, b1, W2l, W2r, b2, W3l, W3r, b3,
           Wc1, bc1, Wc2, bc2):
    src = edge_index[0].astype(jnp.int32)
    dst = edge_index[1].astype(jnp.int32)
    npad = EROWS * CHUNK - NE
    src = jnp.concatenate([src, jnp.zeros((npad,), jnp.int32)]).reshape(EROWS, CHUNK)
    dst = jnp.concatenate([dst, jnp.full((npad,), NN, jnp.int32)]).reshape(EROWS, CHUNK)

    h0 = x[:, :DH]
    h1 = x[:, DH:]
    a0, a1, cnt = _agg_c(h0, h1, src, dst)
    h0p = jnp.pad(h0, ((0, NPAD - NN), (0, 0)))
    h1p = jnp.pad(h1, ((0, NPAD - NN), (0, 0)))
    h0, h1 = _dense(a0, a1, cnt, h0p, h1p, W1l, W1r, b1.reshape(1, DD))
    for Wl, Wr, b in ((W2l, W2r, b2), (W3l, W3r, b3)):
        a0, a1 = _agg_nc(h0, h1, src, dst)
        h0, h1 = _dense(a0, a1, cnt, h0, h1, Wl, Wr, b.reshape(1, DD))

    batch32 = jnp.pad(batch.astype(jnp.int32), (0, NPAD - NN),
                      constant_values=NG).reshape(1, NPAD)
    wc2p = jnp.pad(Wc2, ((0, 0), (0, DD - 2)))
    bc2p = jnp.pad(bc2, (0, DD - 2)).reshape(1, DD)
    out = _pool(h0, h1, batch32, Wc1, bc1.reshape(1, DD // 2), wc2p, bc2p)
    return out[:, :2]


# speedup vs baseline: 2.0413x; 1.2062x over previous
import jax
import jax.numpy as jnp
from jax import lax
from jax.experimental import pallas as pl
from jax.experimental.pallas import tpu as pltpu
from jax.experimental.pallas import tpu_sc as plsc

NN = 10000
NE = 320000
DD = 128
DH = DD // 2
NG = 64

NPAD = 10240
EROWS = 2560
CHUNK = 128
ROWS_PER_TILE = EROWS // 16
STRIPE = NPAD // 16
SUB = STRIPE // CHUNK


IBLK = 16


def _zero_rows(rows):
    def fill_rows(t, carry):
        rows[t // (DH // 16), pl.ds((t % (DH // 16)) * 16, 16)] = (
            jnp.zeros((16,), jnp.float32))
        return carry

    lax.fori_loop(0, CHUNK * (DH // 16), fill_rows, 0)


def _edge_loop(h, srcv, dstv, rows0, rows1, sem0, sem1, ssem0, ssem1,
               acc, cacc, onesv, srcr, dstr, sid, with_counts):
    rows = (rows0, rows1)
    sems = (sem0, sem1)
    ssems = (ssem0, ssem1)

    def outer(jo, carry):
        eb = pl.multiple_of(sid * ROWS_PER_TILE + jo * IBLK, 8)
        pltpu.sync_copy(srcr.at[pl.ds(eb, IBLK)], srcv)
        pltpu.sync_copy(dstr.at[pl.ds(eb, IBLK)], dstv)
        desc = [None, None]
        scat = [None, None]
        desc[0] = pltpu.async_copy(h.at[srcv.at[0]], rows[0], sems[0])
        for t in range(IBLK):
            b = t & 1
            if t + 1 < IBLK:
                if scat[1 - b] is not None:
                    scat[1 - b].wait()
                desc[1 - b] = pltpu.async_copy(
                    h.at[srcv.at[t + 1]], rows[1 - b], sems[1 - b])
            desc[b].wait()
            scat[b] = pltpu.async_copy(
                rows[b], acc.at[dstv.at[t]], ssems[b], add=True)
            if with_counts:
                pltpu.sync_copy(onesv, cacc.at[dstv.at[t]], add=True)
        for b in range(2):
            if scat[b] is not None:
                scat[b].wait()
        return carry

    lax.fori_loop(0, ROWS_PER_TILE // IBLK, outer, 0)


def _agg_body_c(h0, h1, srcr, dstr, agg0o, agg1o, cnto,
                hsp, acc, cacc, srcv, dstv, rows0, rows1, onesv, z16,
                sem0, sem1, ssem0, ssem1):
    cid = lax.axis_index("c")
    sid = lax.axis_index("s")
    r0 = sid * STRIPE

    _zero_rows(rows0)

    def fill_16(i, carry):
        onesv[i, :] = jnp.ones((16,), jnp.float32)
        return carry

    lax.fori_loop(0, CHUNK, fill_16, 0)

    def fill_z16(i, carry):
        z16[i, :] = jnp.zeros((16,), jnp.float32)
        return carry

    lax.fori_loop(0, 64, fill_z16, 0)

    for k in range(SUB):
        pltpu.sync_copy(rows0, acc.at[pl.ds(r0 + k * CHUNK, CHUNK)])
    for k in range(STRIPE // 64):
        pltpu.sync_copy(z16, cacc.at[pl.ds(r0 + k * 64, 64)])

    @pl.when(cid == 0)
    def _():
        pltpu.sync_copy(h0.at[pl.ds(r0, STRIPE)], hsp.at[pl.ds(r0, STRIPE)])

    @pl.when(cid == 1)
    def _():
        pltpu.sync_copy(h1.at[pl.ds(r0, STRIPE)], hsp.at[pl.ds(r0, STRIPE)])

    plsc.subcore_barrier()

    @pl.when(cid == 0)
    def _():
        _edge_loop(hsp, srcv, dstv, rows0, rows1, sem0, sem1, ssem0, ssem1,
                   acc, cacc, onesv, srcr, dstr, sid, True)

    @pl.when(cid == 1)
    def _():
        _edge_loop(hsp, srcv, dstv, rows0, rows1, sem0, sem1, ssem0, ssem1,
                   acc, cacc, onesv, srcr, dstr, sid, False)

    plsc.subcore_barrier()

    for k in range(SUB):
        rk = r0 + k * CHUNK
        pltpu.sync_copy(acc.at[pl.ds(rk, CHUNK)], rows0)

        @pl.when(cid == 0)
        def _():
            pltpu.sync_copy(rows0, agg0o.at[pl.ds(rk, CHUNK)])
            pltpu.sync_copy(cacc.at[pl.ds(rk, CHUNK)], onesv)
            pltpu.sync_copy(onesv, cnto.at[pl.ds(rk, CHUNK)])

        @pl.when(cid == 1)
        def _():
            pltpu.sync_copy(rows0, agg1o.at[pl.ds(rk, CHUNK)])


def _agg_body_nc(h0, h1, srcr, dstr, agg0o, agg1o,
                 hsp, acc, srcv, dstv, rows0, rows1, sem0, sem1, ssem0, ssem1):
    cid = lax.axis_index("c")
    sid = lax.axis_index("s")
    r0 = sid * STRIPE

    _zero_rows(rows0)

    for k in range(SUB):
        pltpu.sync_copy(rows0, acc.at[pl.ds(r0 + k * CHUNK, CHUNK)])

    @pl.when(cid == 0)
    def _():
        pltpu.sync_copy(h0.at[pl.ds(r0, STRIPE)], hsp.at[pl.ds(r0, STRIPE)])

    @pl.when(cid == 1)
    def _():
        pltpu.sync_copy(h1.at[pl.ds(r0, STRIPE)], hsp.at[pl.ds(r0, STRIPE)])

    plsc.subcore_barrier()

    _edge_loop(hsp, srcv, dstv, rows0, rows1, sem0, sem1, ssem0, ssem1,
               acc, None, None, srcr, dstr, sid, False)

    plsc.subcore_barrier()

    for k in range(SUB):
        rk = r0 + k * CHUNK
        pltpu.sync_copy(acc.at[pl.ds(rk, CHUNK)], rows0)

        @pl.when(cid == 0)
        def _():
            pltpu.sync_copy(rows0, agg0o.at[pl.ds(rk, CHUNK)])

        @pl.when(cid == 1)
        def _():
            pltpu.sync_copy(rows0, agg1o.at[pl.ds(rk, CHUNK)])


_SC_MESH = plsc.VectorSubcoreMesh(core_axis_name="c", subcore_axis_name="s")
_SC_PARAMS = pltpu.CompilerParams(use_tc_tiling_on_sc=False)

_agg_c = pl.kernel(
    _agg_body_c,
    out_type=(
        jax.ShapeDtypeStruct((NPAD, DH), jnp.float32),
        jax.ShapeDtypeStruct((NPAD, DH), jnp.float32),
        jax.ShapeDtypeStruct((NPAD, 16), jnp.float32),
    ),
    mesh=_SC_MESH,
    compiler_params=_SC_PARAMS,
    scratch_types=(
        pltpu.VMEM_SHARED((NPAD, DH), jnp.float32),
        pltpu.VMEM_SHARED((NPAD, DH), jnp.float32),
        pltpu.VMEM_SHARED((NPAD, 16), jnp.float32),
        pltpu.VMEM((IBLK, CHUNK), jnp.int32),
        pltpu.VMEM((IBLK, CHUNK), jnp.int32),
        pltpu.VMEM((CHUNK, DH), jnp.float32),
        pltpu.VMEM((CHUNK, DH), jnp.float32),
        pltpu.VMEM((CHUNK, 16), jnp.float32),
        pltpu.VMEM((64, 16), jnp.float32),
        pltpu.SemaphoreType.DMA,
        pltpu.SemaphoreType.DMA,
        pltpu.SemaphoreType.DMA,
        pltpu.SemaphoreType.DMA,
    ),
)

_agg_nc = pl.kernel(
    _agg_body_nc,
    out_type=(
        jax.ShapeDtypeStruct((NPAD, DH), jnp.float32),
        jax.ShapeDtypeStruct((NPAD, DH), jnp.float32),
    ),
    mesh=_SC_MESH,
    compiler_params=_SC_PARAMS,
    scratch_types=(
        pltpu.VMEM_SHARED((NPAD, DH), jnp.float32),
        pltpu.VMEM_SHARED((NPAD, DH), jnp.float32),
        pltpu.VMEM((IBLK, CHUNK), jnp.int32),
        pltpu.VMEM((IBLK, CHUNK), jnp.int32),
        pltpu.VMEM((CHUNK, DH), jnp.float32),
        pltpu.VMEM((CHUNK, DH), jnp.float32),
        pltpu.SemaphoreType.DMA,
        pltpu.SemaphoreType.DMA,
        pltpu.SemaphoreType.DMA,
        pltpu.SemaphoreType.DMA,
    ),
)


def _dense_body(a0_ref, a1_ref, cnt_ref, h0_ref, h1_ref, wl_ref, wr_ref,
                b_ref, o0_ref, o1_ref):
    rc = 1.0 / jnp.maximum(cnt_ref[:, 0:1], 1.0)
    mean = jnp.concatenate([a0_ref[...], a1_ref[...]], axis=1) * rc
    h = jnp.concatenate([h0_ref[...], h1_ref[...]], axis=1)
    o = jnp.dot(mean, wl_ref[...], preferred_element_type=jnp.float32)
    o = o + jnp.dot(h, wr_ref[...], preferred_element_type=jnp.float32)
    o = jnp.maximum(o + b_ref[...], 0.0)
    o0_ref[...] = o[:, :DH]
    o1_ref[...] = o[:, DH:]


_dense = pl.pallas_call(
    _dense_body,
    grid=(10,),
    in_specs=[
        pl.BlockSpec((1024, DH), lambda i: (i, 0)),
        pl.BlockSpec((1024, DH), lambda i: (i, 0)),
        pl.BlockSpec((1024, 16), lambda i: (i, 0)),
        pl.BlockSpec((1024, DH), lambda i: (i, 0)),
        pl.BlockSpec((1024, DH), lambda i: (i, 0)),
        pl.BlockSpec((DD, DD), lambda i: (0, 0)),
        pl.BlockSpec((DD, DD), lambda i: (0, 0)),
        pl.BlockSpec((1, DD), lambda i: (0, 0)),
    ],
    out_specs=[
        pl.BlockSpec((1024, DH), lambda i: (i, 0)),
        pl.BlockSpec((1024, DH), lambda i: (i, 0)),
    ],
    out_shape=[
        jax.ShapeDtypeStruct((NPAD, DH), jnp.float32),
        jax.ShapeDtypeStruct((NPAD, DH), jnp.float32),
    ],
)


def _pool_body(h0_ref, h1_ref, batch_ref, wc1_ref, bc1_ref, wc2_ref,
               bc2_ref, o_ref):
    h = jnp.concatenate([h0_ref[...], h1_ref[...]], axis=1)
    gids = lax.broadcasted_iota(jnp.int32, (NG, NPAD), 0)
    sel = jnp.where(batch_ref[...] == gids, 1.0, 0.0)
    cnts = jnp.sum(sel, axis=1, keepdims=True)
    ps = jnp.dot(sel, h, preferred_element_type=jnp.float32,
                 precision=lax.Precision.HIGHEST)
    pooled = ps / jnp.maximum(cnts, 1.0)
    z = jnp.dot(pooled, wc1_ref[...], preferred_element_type=jnp.float32)
    z = jnp.maximum(z + bc1_ref[...], 0.0)
    o_ref[...] = jnp.dot(z, wc2_ref[...], preferred_element_type=jnp.float32) + bc2_ref[...]


_pool = pl.pallas_call(
    _pool_body,
    out_shape=jax.ShapeDtypeStruct((NG, DD), jnp.float32),
)


def kernel(x, edge_index, batch, W1l, W1r, b1, W2l, W2r, b2, W3l, W3r, b3,
           Wc1, bc1, Wc2, bc2):
    src = edge_index[0].astype(jnp.int32)
    dst = edge_index[1].astype(jnp.int32)
    npad = EROWS * CHUNK - NE
    src = jnp.concatenate([src, jnp.zeros((npad,), jnp.int32)]).reshape(EROWS, CHUNK)
    dst = jnp.concatenate([dst, jnp.full((npad,), NN, jnp.int32)]).reshape(EROWS, CHUNK)

    h0 = x[:, :DH]
    h1 = x[:, DH:]
    h0p = jnp.pad(h0, ((0, NPAD - NN), (0, 0)))
    h1p = jnp.pad(h1, ((0, NPAD - NN), (0, 0)))
    a0, a1, cnt = _agg_c(h0p, h1p, src, dst)
    h0, h1 = _dense(a0, a1, cnt, h0p, h1p, W1l, W1r, b1.reshape(1, DD))
    for Wl, Wr, b in ((W2l, W2r, b2), (W3l, W3r, b3)):
        a0, a1 = _agg_nc(h0, h1, src, dst)
        h0, h1 = _dense(a0, a1, cnt, h0, h1, Wl, Wr, b.reshape(1, DD))

    batch32 = jnp.pad(batch.astype(jnp.int32), (0, NPAD - NN),
                      constant_values=NG).reshape(1, NPAD)
    wc2p = jnp.pad(Wc2, ((0, 0), (0, DD - 2)))
    bc2p = jnp.pad(bc2, (0, DD - 2)).reshape(1, DD)
    out = _pool(h0, h1, batch32, Wc1, bc1.reshape(1, DD // 2), wc2p, bc2p)
    return out[:, :2]


# speedup vs baseline: 2.0430x; 1.0009x over previous
import jax
import jax.numpy as jnp
from jax import lax
from jax.experimental import pallas as pl
from jax.experimental.pallas import tpu as pltpu
from jax.experimental.pallas import tpu_sc as plsc

NN = 10000
NE = 320000
DD = 128
DH = DD // 2
NG = 64

NPAD = 10240
EROWS = 2560
CHUNK = 128
ROWS_PER_TILE = EROWS // 16
STRIPE = NPAD // 16
SUB = STRIPE // CHUNK


IBLK = 16


def _zero_rows(rows):
    def fill_rows(t, carry):
        rows[t // (DH // 16), pl.ds((t % (DH // 16)) * 16, 16)] = (
            jnp.zeros((16,), jnp.float32))
        return carry

    lax.fori_loop(0, CHUNK * (DH // 16), fill_rows, 0)


def _edge_loop(h, srcv, dstv, rows0, rows1, sem0, sem1, ssem0, ssem1,
               acc, cacc, onesv, srcr, dstr, sid, with_counts):
    rows = (rows0, rows1)
    sems = (sem0, sem1)
    ssems = (ssem0, ssem1)

    def outer(jo, carry):
        eb = pl.multiple_of(sid * ROWS_PER_TILE + jo * IBLK, 8)
        pltpu.sync_copy(srcr.at[pl.ds(eb, IBLK)], srcv)
        pltpu.sync_copy(dstr.at[pl.ds(eb, IBLK)], dstv)
        desc = [None, None]
        scat = [None, None]
        desc[0] = pltpu.async_copy(h.at[srcv.at[0]], rows[0], sems[0])
        for t in range(IBLK):
            b = t & 1
            if t + 1 < IBLK:
                if scat[1 - b] is not None:
                    scat[1 - b].wait()
                desc[1 - b] = pltpu.async_copy(
                    h.at[srcv.at[t + 1]], rows[1 - b], sems[1 - b])
            desc[b].wait()
            scat[b] = pltpu.async_copy(
                rows[b], acc.at[dstv.at[t]], ssems[b], add=True)
            if with_counts:
                pltpu.sync_copy(onesv, cacc.at[dstv.at[t]], add=True)
        for b in range(2):
            if scat[b] is not None:
                scat[b].wait()
        return carry

    lax.fori_loop(0, ROWS_PER_TILE // IBLK, outer, 0)


def _agg_body_c(h0, h1, srcr, dstr, agg0o, agg1o, cnto,
                hsp, acc, cacc, srcv, dstv, rows0, rows1, onesv, z16,
                sem0, sem1, ssem0, ssem1):
    cid = lax.axis_index("c")
    sid = lax.axis_index("s")
    r0 = sid * STRIPE

    _zero_rows(rows0)

    def fill_16(i, carry):
        onesv[i, :] = jnp.ones((16,), jnp.float32)
        return carry

    lax.fori_loop(0, CHUNK, fill_16, 0)

    def fill_z16(i, carry):
        z16[i, :] = jnp.zeros((16,), jnp.float32)
        return carry

    lax.fori_loop(0, 64, fill_z16, 0)

    for k in range(SUB):
        pltpu.sync_copy(rows0, acc.at[pl.ds(r0 + k * CHUNK, CHUNK)])
    for k in range(STRIPE // 64):
        pltpu.sync_copy(z16, cacc.at[pl.ds(r0 + k * 64, 64)])

    @pl.when(cid == 0)
    def _():
        pltpu.sync_copy(h0.at[pl.ds(r0, STRIPE)], hsp.at[pl.ds(r0, STRIPE)])

    @pl.when(cid == 1)
    def _():
        pltpu.sync_copy(h1.at[pl.ds(r0, STRIPE)], hsp.at[pl.ds(r0, STRIPE)])

    plsc.subcore_barrier()

    @pl.when(cid == 0)
    def _():
        _edge_loop(hsp, srcv, dstv, rows0, rows1, sem0, sem1, ssem0, ssem1,
                   acc, cacc, onesv, srcr, dstr, sid, True)

    @pl.when(cid == 1)
    def _():
        _edge_loop(hsp, srcv, dstv, rows0, rows1, sem0, sem1, ssem0, ssem1,
                   acc, cacc, onesv, srcr, dstr, sid, False)

    plsc.subcore_barrier()

    @pl.when(cid == 0)
    def _():
        pltpu.sync_copy(acc.at[pl.ds(r0, STRIPE)], agg0o.at[pl.ds(r0, STRIPE)])
        pltpu.sync_copy(cacc.at[pl.ds(r0, STRIPE)], cnto.at[pl.ds(r0, STRIPE)])

    @pl.when(cid == 1)
    def _():
        pltpu.sync_copy(acc.at[pl.ds(r0, STRIPE)], agg1o.at[pl.ds(r0, STRIPE)])


def _agg_body_nc(h0, h1, srcr, dstr, agg0o, agg1o,
                 hsp, acc, srcv, dstv, rows0, rows1, sem0, sem1, ssem0, ssem1):
    cid = lax.axis_index("c")
    sid = lax.axis_index("s")
    r0 = sid * STRIPE

    _zero_rows(rows0)

    for k in range(SUB):
        pltpu.sync_copy(rows0, acc.at[pl.ds(r0 + k * CHUNK, CHUNK)])

    @pl.when(cid == 0)
    def _():
        pltpu.sync_copy(h0.at[pl.ds(r0, STRIPE)], hsp.at[pl.ds(r0, STRIPE)])

    @pl.when(cid == 1)
    def _():
        pltpu.sync_copy(h1.at[pl.ds(r0, STRIPE)], hsp.at[pl.ds(r0, STRIPE)])

    plsc.subcore_barrier()

    _edge_loop(hsp, srcv, dstv, rows0, rows1, sem0, sem1, ssem0, ssem1,
               acc, None, None, srcr, dstr, sid, False)

    plsc.subcore_barrier()

    @pl.when(cid == 0)
    def _():
        pltpu.sync_copy(acc.at[pl.ds(r0, STRIPE)], agg0o.at[pl.ds(r0, STRIPE)])

    @pl.when(cid == 1)
    def _():
        pltpu.sync_copy(acc.at[pl.ds(r0, STRIPE)], agg1o.at[pl.ds(r0, STRIPE)])


_SC_MESH = plsc.VectorSubcoreMesh(core_axis_name="c", subcore_axis_name="s")
_SC_PARAMS = pltpu.CompilerParams(use_tc_tiling_on_sc=False)

_agg_c = pl.kernel(
    _agg_body_c,
    out_type=(
        jax.ShapeDtypeStruct((NPAD, DH), jnp.float32),
        jax.ShapeDtypeStruct((NPAD, DH), jnp.float32),
        jax.ShapeDtypeStruct((NPAD, 16), jnp.float32),
    ),
    mesh=_SC_MESH,
    compiler_params=_SC_PARAMS,
    scratch_types=(
        pltpu.VMEM_SHARED((NPAD, DH), jnp.float32),
        pltpu.VMEM_SHARED((NPAD, DH), jnp.float32),
        pltpu.VMEM_SHARED((NPAD, 16), jnp.float32),
        pltpu.VMEM((IBLK, CHUNK), jnp.int32),
        pltpu.VMEM((IBLK, CHUNK), jnp.int32),
        pltpu.VMEM((CHUNK, DH), jnp.float32),
        pltpu.VMEM((CHUNK, DH), jnp.float32),
        pltpu.VMEM((CHUNK, 16), jnp.float32),
        pltpu.VMEM((64, 16), jnp.float32),
        pltpu.SemaphoreType.DMA,
        pltpu.SemaphoreType.DMA,
        pltpu.SemaphoreType.DMA,
        pltpu.SemaphoreType.DMA,
    ),
)

_agg_nc = pl.kernel(
    _agg_body_nc,
    out_type=(
        jax.ShapeDtypeStruct((NPAD, DH), jnp.float32),
        jax.ShapeDtypeStruct((NPAD, DH), jnp.float32),
    ),
    mesh=_SC_MESH,
    compiler_params=_SC_PARAMS,
    scratch_types=(
        pltpu.VMEM_SHARED((NPAD, DH), jnp.float32),
        pltpu.VMEM_SHARED((NPAD, DH), jnp.float32),
        pltpu.VMEM((IBLK, CHUNK), jnp.int32),
        pltpu.VMEM((IBLK, CHUNK), jnp.int32),
        pltpu.VMEM((CHUNK, DH), jnp.float32),
        pltpu.VMEM((CHUNK, DH), jnp.float32),
        pltpu.SemaphoreType.DMA,
        pltpu.SemaphoreType.DMA,
        pltpu.SemaphoreType.DMA,
        pltpu.SemaphoreType.DMA,
    ),
)


def _dense_body(a0_ref, a1_ref, cnt_ref, h0_ref, h1_ref, wl_ref, wr_ref,
                b_ref, o0_ref, o1_ref):
    rc = 1.0 / jnp.maximum(cnt_ref[:, 0:1], 1.0)
    mean = jnp.concatenate([a0_ref[...], a1_ref[...]], axis=1) * rc
    h = jnp.concatenate([h0_ref[...], h1_ref[...]], axis=1)
    o = jnp.dot(mean, wl_ref[...], preferred_element_type=jnp.float32)
    o = o + jnp.dot(h, wr_ref[...], preferred_element_type=jnp.float32)
    o = jnp.maximum(o + b_ref[...], 0.0)
    o0_ref[...] = o[:, :DH]
    o1_ref[...] = o[:, DH:]


_dense = pl.pallas_call(
    _dense_body,
    grid=(10,),
    in_specs=[
        pl.BlockSpec((1024, DH), lambda i: (i, 0)),
        pl.BlockSpec((1024, DH), lambda i: (i, 0)),
        pl.BlockSpec((1024, 16), lambda i: (i, 0)),
        pl.BlockSpec((1024, DH), lambda i: (i, 0)),
        pl.BlockSpec((1024, DH), lambda i: (i, 0)),
        pl.BlockSpec((DD, DD), lambda i: (0, 0)),
        pl.BlockSpec((DD, DD), lambda i: (0, 0)),
        pl.BlockSpec((1, DD), lambda i: (0, 0)),
    ],
    out_specs=[
        pl.BlockSpec((1024, DH), lambda i: (i, 0)),
        pl.BlockSpec((1024, DH), lambda i: (i, 0)),
    ],
    out_shape=[
        jax.ShapeDtypeStruct((NPAD, DH), jnp.float32),
        jax.ShapeDtypeStruct((NPAD, DH), jnp.float32),
    ],
)


def _pool_body(h0_ref, h1_ref, batch_ref, wc1_ref, bc1_ref, wc2_ref,
               bc2_ref, o_ref):
    h = jnp.concatenate([h0_ref[...], h1_ref[...]], axis=1)
    gids = lax.broadcasted_iota(jnp.int32, (NG, NPAD), 0)
    sel = jnp.where(batch_ref[...] == gids, 1.0, 0.0)
    cnts = jnp.sum(sel, axis=1, keepdims=True)
    ps = jnp.dot(sel, h, preferred_element_type=jnp.float32,
                 precision=lax.Precision.HIGHEST)
    pooled = ps / jnp.maximum(cnts, 1.0)
    z = jnp.dot(pooled, wc1_ref[...], preferred_element_type=jnp.float32)
    z = jnp.maximum(z + bc1_ref[...], 0.0)
    o_ref[...] = jnp.dot(z, wc2_ref[...], preferred_element_type=jnp.float32) + bc2_ref[...]


_pool = pl.pallas_call(
    _pool_body,
    out_shape=jax.ShapeDtypeStruct((NG, DD), jnp.float32),
)


def kernel(x, edge_index, batch, W1l, W1r, b1, W2l, W2r, b2, W3l, W3r, b3,
           Wc1, bc1, Wc2, bc2):
    src = edge_index[0].astype(jnp.int32)
    dst = edge_index[1].astype(jnp.int32)
    npad = EROWS * CHUNK - NE
    src = jnp.concatenate([src, jnp.zeros((npad,), jnp.int32)]).reshape(EROWS, CHUNK)
    dst = jnp.concatenate([dst, jnp.full((npad,), NN, jnp.int32)]).reshape(EROWS, CHUNK)

    h0 = x[:, :DH]
    h1 = x[:, DH:]
    h0p = jnp.pad(h0, ((0, NPAD - NN), (0, 0)))
    h1p = jnp.pad(h1, ((0, NPAD - NN), (0, 0)))
    a0, a1, cnt = _agg_c(h0p, h1p, src, dst)
    h0, h1 = _dense(a0, a1, cnt, h0p, h1p, W1l, W1r, b1.reshape(1, DD))
    for Wl, Wr, b in ((W2l, W2r, b2), (W3l, W3r, b3)):
        a0, a1 = _agg_nc(h0, h1, src, dst)
        h0, h1 = _dense(a0, a1, cnt, h0, h1, Wl, Wr, b.reshape(1, DD))

    batch32 = jnp.pad(batch.astype(jnp.int32), (0, NPAD - NN),
                      constant_values=NG).reshape(1, NPAD)
    wc2p = jnp.pad(Wc2, ((0, 0), (0, DD - 2)))
    bc2p = jnp.pad(bc2, (0, DD - 2)).reshape(1, DD)
    out = _pool(h0, h1, batch32, Wc1, bc1.reshape(1, DD // 2), wc2p, bc2p)
    return out[:, :2]
